# Initial kernel scaffold; baseline (speedup 1.0000x reference)
import jax
import jax.numpy as jnp
from jax.experimental import pallas as pl


def kernel(pred, target, epoch_i):
    raise NotImplementedError("write your pallas kernel here")


# speedup vs baseline: 14.8322x; 14.8322x over previous
import functools

import jax
import jax.numpy as jnp
from jax import lax
from jax.experimental import pallas as pl
from jax.experimental.pallas import tpu as pltpu

_THRESH = 0.7
_MIN_KEPT = 100000

_N, _C, _H, _W = 2, 150, 512, 512
_HB = 16


def _stats_body(pred_ref, tgt_ref, p_ref, nll_ref):
    x = pred_ref[0]
    t = tgt_ref[0]
    m = jnp.max(x, axis=0)
    s = jnp.sum(jnp.exp(x - m[None]), axis=0)
    cls = lax.broadcasted_iota(jnp.int32, x.shape, 0)
    xt = jnp.sum(jnp.where(cls == t[None], x, 0.0), axis=0)
    p_ref[0] = jnp.exp(xt - m) / s
    nll_ref[0] = jnp.log(s) + m - xt


def _select_body(p_ref, nll_ref, out_ref):
    p = p_ref[...]
    bits = lax.bitcast_convert_type(p, jnp.int32)
    k = jnp.int32(_MIN_KEPT)

    def step(i, prefix):
        b = 30 - i
        cand = prefix + (jnp.int32(1) << b)
        cnt = jnp.sum((bits < cand).astype(jnp.int32))
        return jnp.where(cnt >= k, prefix, cand)

    vbits = lax.fori_loop(0, 31, step, jnp.int32(0))
    thr = lax.bitcast_convert_type(vbits, jnp.float32)
    threshold = jnp.maximum(thr, jnp.float32(_THRESH))

    kept = p <= threshold
    cnt = jnp.sum(kept.astype(jnp.float32))
    ssum = jnp.sum(jnp.where(kept, nll_ref[...], 0.0))
    out_ref[0, 0] = ssum / jnp.maximum(cnt, 1.0)


@jax.jit
def _ohem(pred, target):
    p, nll = pl.pallas_call(
        _stats_body,
        grid=(_N, _H // _HB),
        in_specs=[
            pl.BlockSpec((1, _C, _HB, _W), lambda n, h: (n, 0, h, 0)),
            pl.BlockSpec((1, _HB, _W), lambda n, h: (n, h, 0)),
        ],
        out_specs=[
            pl.BlockSpec((1, _HB, _W), lambda n, h: (n, h, 0)),
            pl.BlockSpec((1, _HB, _W), lambda n, h: (n, h, 0)),
        ],
        out_shape=[
            jax.ShapeDtypeStruct((_N, _H, _W), jnp.float32),
            jax.ShapeDtypeStruct((_N, _H, _W), jnp.float32),
        ],
        compiler_params=pltpu.CompilerParams(
            dimension_semantics=("parallel", "parallel"),
        ),
    )(pred, target)

    p2 = p.reshape(_H, _N * _W)
    nll2 = nll.reshape(_H, _N * _W)
    loss = pl.pallas_call(
        _select_body,
        in_specs=[
            pl.BlockSpec(memory_space=pltpu.MemorySpace.VMEM),
            pl.BlockSpec(memory_space=pltpu.MemorySpace.VMEM),
        ],
        out_specs=pl.BlockSpec(memory_space=pltpu.MemorySpace.SMEM),
        out_shape=jax.ShapeDtypeStruct((1, 1), jnp.float32),
    )(p2, nll2)
    return loss[0, 0]


def kernel(pred, target, epoch_i):
    return _ohem(pred, target)


# speedup vs baseline: 17.0245x; 1.1478x over previous
import functools

import jax
import jax.numpy as jnp
from jax import lax
from jax.experimental import pallas as pl
from jax.experimental.pallas import tpu as pltpu

_THRESH = 0.7
_MIN_KEPT = 100000

_N, _C, _H, _W = 2, 150, 512, 512
_HB = 16
_STEPS = _N * (_H // _HB)


def _fused_body(pred_ref, tgt_ref, p_ref, nll_ref, loss_ref, cnt_ref, acc_ref):
    step = pl.program_id(0) * (_H // _HB) + pl.program_id(1)

    @pl.when(step == 0)
    def _init():
        acc_ref[0] = 0.0
        acc_ref[1] = 0.0

    x = pred_ref[0]
    t = tgt_ref[0]
    m = jnp.max(x, axis=0)
    s = jnp.sum(jnp.exp(x - m[None]), axis=0)
    cls = lax.broadcasted_iota(jnp.int32, x.shape, 0)
    xt = jnp.sum(jnp.where(cls == t[None], x, 0.0), axis=0)
    p = jnp.exp(xt - m) / s
    nll = jnp.log(s) + m - xt
    p_ref[0] = p
    nll_ref[0] = nll

    mask = p <= _THRESH
    acc_ref[0] += jnp.sum(mask.astype(jnp.float32))
    acc_ref[1] += jnp.sum(jnp.where(mask, nll, 0.0))

    @pl.when(step == _STEPS - 1)
    def _fin():
        cnt = acc_ref[0]
        loss_ref[0, 0] = acc_ref[1] / jnp.maximum(cnt, 1.0)
        cnt_ref[0, 0] = cnt


def _select_body(p_ref, nll_ref, out_ref):
    p = p_ref[...]
    bits = lax.bitcast_convert_type(p, jnp.int32)
    k = jnp.int32(_MIN_KEPT)

    def step(i, prefix):
        b = 30 - i
        cand = prefix + (jnp.int32(1) << b)
        cnt = jnp.sum((bits < cand).astype(jnp.int32))
        return jnp.where(cnt >= k, prefix, cand)

    vbits = lax.fori_loop(0, 31, step, jnp.int32(0))
    thr = lax.bitcast_convert_type(vbits, jnp.float32)
    threshold = jnp.maximum(thr, jnp.float32(_THRESH))

    kept = p <= threshold
    cnt = jnp.sum(kept.astype(jnp.float32))
    ssum = jnp.sum(jnp.where(kept, nll_ref[...], 0.0))
    out_ref[0, 0] = ssum / jnp.maximum(cnt, 1.0)


def _rare_select(p, nll):
    loss = pl.pallas_call(
        _select_body,
        in_specs=[
            pl.BlockSpec(memory_space=pltpu.MemorySpace.VMEM),
            pl.BlockSpec(memory_space=pltpu.MemorySpace.VMEM),
        ],
        out_specs=pl.BlockSpec(memory_space=pltpu.MemorySpace.SMEM),
        out_shape=jax.ShapeDtypeStruct((1, 1), jnp.float32),
    )(p, nll)
    return loss[0, 0]


@jax.jit
def _ohem(pred, target):
    p, nll, loss_fast, cnt = pl.pallas_call(
        _fused_body,
        grid=(_N, _H // _HB),
        in_specs=[
            pl.BlockSpec((1, _C, _HB, _W), lambda n, h: (n, 0, h, 0)),
            pl.BlockSpec((1, _HB, _W), lambda n, h: (n, h, 0)),
        ],
        out_specs=[
            pl.BlockSpec((1, _HB, _W), lambda n, h: (n, h, 0)),
            pl.BlockSpec((1, _HB, _W), lambda n, h: (n, h, 0)),
            pl.BlockSpec(memory_space=pltpu.MemorySpace.SMEM),
            pl.BlockSpec(memory_space=pltpu.MemorySpace.SMEM),
        ],
        out_shape=[
            jax.ShapeDtypeStruct((_N, _H, _W), jnp.float32),
            jax.ShapeDtypeStruct((_N, _H, _W), jnp.float32),
            jax.ShapeDtypeStruct((1, 1), jnp.float32),
            jax.ShapeDtypeStruct((1, 1), jnp.float32),
        ],
        scratch_shapes=[pltpu.SMEM((2,), jnp.float32)],
        compiler_params=pltpu.CompilerParams(
            dimension_semantics=("arbitrary", "arbitrary"),
        ),
    )(pred, target)

    return lax.cond(
        cnt[0, 0] >= jnp.float32(_MIN_KEPT),
        lambda: loss_fast[0, 0],
        lambda: _rare_select(p, nll),
    )


def kernel(pred, target, epoch_i):
    return _ohem(pred, target)


# speedup vs baseline: 20.6956x; 1.2156x over previous
import functools

import jax
import jax.numpy as jnp
from jax import lax
from jax.experimental import pallas as pl
from jax.experimental.pallas import tpu as pltpu

_THRESH = 0.7
_MIN_KEPT = 100000

_N, _C, _H, _W = 2, 150, 512, 512
_HB = 32
_STEPS = _N * (_H // _HB)


def _fused_body(pred_ref, tgt_ref, p_ref, nll_ref, loss_ref, cnt_ref,
                acc_cnt_ref, acc_sum_ref):
    step = pl.program_id(0) * (_H // _HB) + pl.program_id(1)

    @pl.when(step == 0)
    def _init():
        acc_cnt_ref[...] = jnp.zeros_like(acc_cnt_ref)
        acc_sum_ref[...] = jnp.zeros_like(acc_sum_ref)

    x = pred_ref[0]
    t = tgt_ref[0]
    m = jnp.max(x, axis=0)
    s = jnp.sum(jnp.exp(x - m[None]), axis=0)
    cls = lax.broadcasted_iota(jnp.int32, x.shape, 0)
    xt = jnp.sum(jnp.where(cls == t[None], x, 0.0), axis=0)
    p = jnp.exp(xt - m) / s
    nll = jnp.log(s) + m - xt
    p_ref[0] = p
    nll_ref[0] = nll

    mask = p <= _THRESH
    acc_cnt_ref[...] += mask.astype(jnp.float32)
    acc_sum_ref[...] += jnp.where(mask, nll, 0.0)

    @pl.when(step == _STEPS - 1)
    def _fin():
        cnt = jnp.sum(acc_cnt_ref[...])
        loss_ref[0, 0] = jnp.sum(acc_sum_ref[...]) / jnp.maximum(cnt, 1.0)
        cnt_ref[0, 0] = cnt


def _select_body(p_ref, nll_ref, out_ref):
    p = p_ref[...]
    bits = lax.bitcast_convert_type(p, jnp.int32)
    k = jnp.int32(_MIN_KEPT)

    def step(i, prefix):
        b = 30 - i
        cand = prefix + (jnp.int32(1) << b)
        cnt = jnp.sum((bits < cand).astype(jnp.int32))
        return jnp.where(cnt >= k, prefix, cand)

    vbits = lax.fori_loop(0, 31, step, jnp.int32(0))
    thr = lax.bitcast_convert_type(vbits, jnp.float32)
    threshold = jnp.maximum(thr, jnp.float32(_THRESH))

    kept = p <= threshold
    cnt = jnp.sum(kept.astype(jnp.float32))
    ssum = jnp.sum(jnp.where(kept, nll_ref[...], 0.0))
    out_ref[0, 0] = ssum / jnp.maximum(cnt, 1.0)


def _rare_select(p, nll):
    loss = pl.pallas_call(
        _select_body,
        in_specs=[
            pl.BlockSpec(memory_space=pltpu.MemorySpace.VMEM),
            pl.BlockSpec(memory_space=pltpu.MemorySpace.VMEM),
        ],
        out_specs=pl.BlockSpec(memory_space=pltpu.MemorySpace.SMEM),
        out_shape=jax.ShapeDtypeStruct((1, 1), jnp.float32),
    )(p, nll)
    return loss[0, 0]


@jax.jit
def _ohem(pred, target):
    p, nll, loss_fast, cnt = pl.pallas_call(
        _fused_body,
        grid=(_N, _H // _HB),
        in_specs=[
            pl.BlockSpec((1, _C, _HB, _W), lambda n, h: (n, 0, h, 0)),
            pl.BlockSpec((1, _HB, _W), lambda n, h: (n, h, 0)),
        ],
        out_specs=[
            pl.BlockSpec((1, _HB, _W), lambda n, h: (n, h, 0)),
            pl.BlockSpec((1, _HB, _W), lambda n, h: (n, h, 0)),
            pl.BlockSpec(memory_space=pltpu.MemorySpace.SMEM),
            pl.BlockSpec(memory_space=pltpu.MemorySpace.SMEM),
        ],
        out_shape=[
            jax.ShapeDtypeStruct((_N, _H, _W), jnp.float32),
            jax.ShapeDtypeStruct((_N, _H, _W), jnp.float32),
            jax.ShapeDtypeStruct((1, 1), jnp.float32),
            jax.ShapeDtypeStruct((1, 1), jnp.float32),
        ],
        scratch_shapes=[
            pltpu.VMEM((_HB, _W), jnp.float32),
            pltpu.VMEM((_HB, _W), jnp.float32),
        ],
        compiler_params=pltpu.CompilerParams(
            dimension_semantics=("arbitrary", "arbitrary"),
        ),
    )(pred, target)

    return lax.cond(
        cnt[0, 0] >= jnp.float32(_MIN_KEPT),
        lambda: loss_fast[0, 0],
        lambda: _rare_select(p, nll),
    )


def kernel(pred, target, epoch_i):
    return _ohem(pred, target)


# speedup vs baseline: 21.7484x; 1.0509x over previous
import functools

import jax
import jax.numpy as jnp
from jax import lax
from jax.experimental import pallas as pl
from jax.experimental.pallas import tpu as pltpu

_THRESH = 0.7
_MIN_KEPT = 100000

_N, _C, _H, _W = 2, 150, 512, 512
_HB = 64
_STEPS = _N * (_H // _HB)


def _fused_body(pred_ref, tgt_ref, p_ref, nll_ref, loss_ref, cnt_ref,
                acc_cnt_ref, acc_sum_ref):
    step = pl.program_id(0) * (_H // _HB) + pl.program_id(1)

    @pl.when(step == 0)
    def _init():
        acc_cnt_ref[...] = jnp.zeros_like(acc_cnt_ref)
        acc_sum_ref[...] = jnp.zeros_like(acc_sum_ref)

    x = pred_ref[0]
    t = tgt_ref[0]
    m = jnp.max(x, axis=0)
    s = jnp.sum(jnp.exp(x - m[None]), axis=0)
    cls = lax.broadcasted_iota(jnp.int32, x.shape, 0)
    xt = jnp.sum(jnp.where(cls == t[None], x, 0.0), axis=0)
    p = jnp.exp(xt - m) / s
    nll = jnp.log(s) + m - xt
    p_ref[0] = p
    nll_ref[0] = nll

    mask = p <= _THRESH
    acc_cnt_ref[...] += mask.astype(jnp.float32)
    acc_sum_ref[...] += jnp.where(mask, nll, 0.0)

    @pl.when(step == _STEPS - 1)
    def _fin():
        cnt = jnp.sum(acc_cnt_ref[...])
        loss_ref[0, 0] = jnp.sum(acc_sum_ref[...]) / jnp.maximum(cnt, 1.0)
        cnt_ref[0, 0] = cnt


def _select_body(p_ref, nll_ref, out_ref):
    p = p_ref[...]
    bits = lax.bitcast_convert_type(p, jnp.int32)
    k = jnp.int32(_MIN_KEPT)

    def step(i, prefix):
        b = 30 - i
        cand = prefix + (jnp.int32(1) << b)
        cnt = jnp.sum((bits < cand).astype(jnp.int32))
        return jnp.where(cnt >= k, prefix, cand)

    vbits = lax.fori_loop(0, 31, step, jnp.int32(0))
    thr = lax.bitcast_convert_type(vbits, jnp.float32)
    threshold = jnp.maximum(thr, jnp.float32(_THRESH))

    kept = p <= threshold
    cnt = jnp.sum(kept.astype(jnp.float32))
    ssum = jnp.sum(jnp.where(kept, nll_ref[...], 0.0))
    out_ref[0, 0] = ssum / jnp.maximum(cnt, 1.0)


def _rare_select(p, nll):
    loss = pl.pallas_call(
        _select_body,
        in_specs=[
            pl.BlockSpec(memory_space=pltpu.MemorySpace.VMEM),
            pl.BlockSpec(memory_space=pltpu.MemorySpace.VMEM),
        ],
        out_specs=pl.BlockSpec(memory_space=pltpu.MemorySpace.SMEM),
        out_shape=jax.ShapeDtypeStruct((1, 1), jnp.float32),
    )(p, nll)
    return loss[0, 0]


@jax.jit
def _ohem(pred, target):
    p, nll, loss_fast, cnt = pl.pallas_call(
        _fused_body,
        grid=(_N, _H // _HB),
        in_specs=[
            pl.BlockSpec((1, _C, _HB, _W), lambda n, h: (n, 0, h, 0)),
            pl.BlockSpec((1, _HB, _W), lambda n, h: (n, h, 0)),
        ],
        out_specs=[
            pl.BlockSpec((1, _HB, _W), lambda n, h: (n, h, 0)),
            pl.BlockSpec((1, _HB, _W), lambda n, h: (n, h, 0)),
            pl.BlockSpec(memory_space=pltpu.MemorySpace.SMEM),
            pl.BlockSpec(memory_space=pltpu.MemorySpace.SMEM),
        ],
        out_shape=[
            jax.ShapeDtypeStruct((_N, _H, _W), jnp.float32),
            jax.ShapeDtypeStruct((_N, _H, _W), jnp.float32),
            jax.ShapeDtypeStruct((1, 1), jnp.float32),
            jax.ShapeDtypeStruct((1, 1), jnp.float32),
        ],
        scratch_shapes=[
            pltpu.VMEM((_HB, _W), jnp.float32),
            pltpu.VMEM((_HB, _W), jnp.float32),
        ],
        compiler_params=pltpu.CompilerParams(
            dimension_semantics=("arbitrary", "arbitrary"),
        ),
    )(pred, target)

    return lax.cond(
        cnt[0, 0] >= jnp.float32(_MIN_KEPT),
        lambda: loss_fast[0, 0],
        lambda: _rare_select(p, nll),
    )


def kernel(pred, target, epoch_i):
    return _ohem(pred, target)
